# baseline (speedup 1.0000x reference)
import functools

import jax
import jax.numpy as jnp
from jax import lax
from jax.experimental import pallas as pl
from jax.experimental.pallas import tpu as pltpu
from jax.experimental.pallas import tpu_sc as plsc

T = 100
E = 64
B = 4096

_info = plsc.get_sparse_core_info()
_NC, _NS = _info.num_cores, _info.num_subcores
_NW = _NC * _NS
_BPW = B // _NW
_K = 10


@functools.partial(
    pl.kernel,
    out_type=jax.ShapeDtypeStruct((B, T, E), jnp.float32),
    mesh=plsc.VectorSubcoreMesh(core_axis_name="c", subcore_axis_name="s"),
    scratch_types=[
        pltpu.VMEM((_BPW,), jnp.int32),
        pltpu.VMEM((_BPW, E), jnp.float32),
        pltpu.VMEM((_BPW, _K, E), jnp.float32),
        pltpu.VMEM_SHARED((_NS, _BPW, 1, E), jnp.float32),
        pltpu.SemaphoreType.DMA,
    ],
    compiler_params=pltpu.CompilerParams(use_tc_tiling_on_sc=False),
)
def _embed_repeat(x_hbm, table_hbm, out_hbm, idx_v, rows_v, buf_v, spm, sem):
    wid = lax.axis_index("s") * _NC + lax.axis_index("c")
    sid = lax.axis_index("s")
    base = wid * _BPW
    pltpu.sync_copy(x_hbm.at[pl.ds(base, _BPW)], idx_v)
    pltpu.async_copy(table_hbm.at[idx_v], rows_v, sem).wait()
    pltpu.sync_copy(rows_v, spm.at[sid, pl.ds(0, _BPW), 0])
    reps = [
        pltpu.async_copy(
            spm.at[sid],
            buf_v.at[pl.ds(0, _BPW), pl.ds(m, 1), pl.ds(0, E)],
            sem,
        )
        for m in range(_K)
    ]
    for g in reps:
        g.wait()
    copies = [
        pltpu.async_copy(
            buf_v,
            out_hbm.at[pl.ds(base, _BPW), pl.ds(j * _K, _K), pl.ds(0, E)],
            sem,
        )
        for j in range(T // _K)
    ]
    for c in copies:
        c.wait()


def kernel(x, ue_weight):
    return _embed_repeat(x.astype(jnp.int32), ue_weight)


# speedup vs baseline: 1.3783x; 1.3783x over previous
import functools

import jax
import jax.numpy as jnp
from jax import lax
from jax.experimental import pallas as pl
from jax.experimental.pallas import tpu as pltpu
from jax.experimental.pallas import tpu_sc as plsc

T = 100
E = 64
B = 4096

_info = plsc.get_sparse_core_info()
_NC, _NS, _L = _info.num_cores, _info.num_subcores, _info.num_lanes
_NW = _NC * _NS
_BPW = B // _NW
_EH = E // 8


@functools.partial(
    pl.kernel,
    out_type=jax.ShapeDtypeStruct((T, _EH, B // _BPW, 8, _BPW), jnp.float32),
    mesh=plsc.VectorSubcoreMesh(core_axis_name="c", subcore_axis_name="s"),
    scratch_types=[
        pltpu.VMEM((_BPW,), jnp.int32),
        pltpu.VMEM((_BPW, E), jnp.float32),
        pltpu.VMEM((1, _EH, 1, 8, _BPW), jnp.float32),
        pltpu.SemaphoreType.DMA,
    ],
    compiler_params=pltpu.CompilerParams(
        use_tc_tiling_on_sc=False, needs_layout_passes=False
    ),
)
def _embed_repeat(x_hbm, table_hbm, out_hbm, idx_v, rows_v, blk_v, sem):
    wid = lax.axis_index("s") * _NC + lax.axis_index("c")
    base = wid * _BPW
    pltpu.sync_copy(x_hbm.at[pl.ds(base, _BPW)], idx_v)
    pltpu.async_copy(table_hbm.at[idx_v], rows_v, sem).wait()
    row_ids = [lax.iota(jnp.int32, _L) + g * _L for g in range(_BPW // _L)]
    for eh in range(_EH):
        for el in range(8):
            col_id = jnp.full((_L,), eh * 8 + el, jnp.int32)
            for g in range(_BPW // _L):
                v = plsc.load_gather(rows_v, [row_ids[g], col_id])
                blk_v[0, eh, 0, el, pl.ds(g * _L, _L)] = v
    copies = [
        pltpu.async_copy(
            blk_v,
            out_hbm.at[
                pl.ds(t, 1),
                pl.ds(0, _EH),
                pl.ds(wid, 1),
                pl.ds(0, 8),
                pl.ds(0, _BPW),
            ],
            sem,
        )
        for t in range(T)
    ]
    for c in copies:
        c.wait()


def kernel(x, ue_weight):
    out = _embed_repeat(x.astype(jnp.int32), ue_weight)
    return out.transpose(2, 4, 0, 1, 3).reshape(B, T, E)


# speedup vs baseline: 6.1371x; 4.4528x over previous
import functools

import jax
import jax.numpy as jnp
from jax import lax
from jax.experimental import pallas as pl
from jax.experimental.pallas import tpu as pltpu
from jax.experimental.pallas import tpu_sc as plsc

T = 100
E = 64
B = 4096

_info = plsc.get_sparse_core_info()
_NC, _NS, _L = _info.num_cores, _info.num_subcores, _info.num_lanes
_NW = _NC * _NS
_BPW = B // _NW
_EH = E // 8
_NG = _BPW // _L


@functools.partial(
    pl.kernel,
    out_type=jax.ShapeDtypeStruct((T, _EH, B // _BPW, 8, _BPW), jnp.float32),
    mesh=plsc.VectorSubcoreMesh(core_axis_name="c", subcore_axis_name="s"),
    scratch_types=[
        pltpu.VMEM((_BPW,), jnp.int32),
        pltpu.VMEM((_L, 8, 128), jnp.float32),
        pltpu.VMEM((1, _EH, 1, 8, _BPW), jnp.float32),
        pltpu.SemaphoreType.DMA,
    ],
    compiler_params=pltpu.CompilerParams(
        use_tc_tiling_on_sc=True, needs_layout_passes=False
    ),
)
def _embed_repeat(x_hbm, tblt_hbm, out_hbm, idx_v, st_v, blk_v, sem):
    wid = lax.axis_index("s") * _NC + lax.axis_index("c")
    base = wid * _BPW
    pltpu.sync_copy(x_hbm.at[pl.ds(base, _BPW)], idx_v)
    lane_iota = lax.iota(jnp.int32, _L)

    def group_body(g, carry):
        v = idx_v[pl.ds(g * _L, _L)]
        rt = lax.shift_right_logical(v, 7)
        lane = lax.bitwise_and(v, jnp.full((_L,), 127, jnp.int32))
        rts = [
            jnp.sum(jnp.where(lane_iota == j, rt, jnp.zeros((_L,), jnp.int32)))
            for j in range(_L)
        ]
        for eh in range(_EH):
            fetches = [
                pltpu.async_copy(
                    tblt_hbm.at[pl.ds(eh * 8, 8), pl.ds(rts[j] * 128, 128)],
                    st_v.at[j],
                    sem,
                )
                for j in range(_L)
            ]
            for f in fetches:
                f.wait()
            for el in range(8):
                vals = plsc.load_gather(
                    st_v, [lane_iota, jnp.full((_L,), el, jnp.int32), lane]
                )
                blk_v[0, eh, 0, el, pl.ds(g * _L, _L)] = vals
        return carry

    lax.fori_loop(0, _NG, group_body, 0)
    copies = [
        pltpu.async_copy(
            blk_v,
            out_hbm.at[
                pl.ds(t, 1),
                pl.ds(0, _EH),
                pl.ds(wid, 1),
                pl.ds(0, 8),
                pl.ds(0, _BPW),
            ],
            sem,
        )
        for t in range(T)
    ]
    for c in copies:
        c.wait()


def kernel(x, ue_weight):
    out = _embed_repeat(x.astype(jnp.int32), ue_weight.T)
    return out.transpose(2, 4, 0, 1, 3).reshape(B, T, E)


# speedup vs baseline: 7.4953x; 1.2213x over previous
import functools

import jax
import jax.numpy as jnp
from jax import lax
from jax.experimental import pallas as pl
from jax.experimental.pallas import tpu as pltpu
from jax.experimental.pallas import tpu_sc as plsc

T = 100
E = 64
B = 4096

_info = plsc.get_sparse_core_info()
_NC, _NS, _L = _info.num_cores, _info.num_subcores, _info.num_lanes
_NW = _NC * _NS
_BPW = B // _NW
_EH = E // 8
_NG = _BPW // _L


@functools.partial(
    pl.kernel,
    out_type=jax.ShapeDtypeStruct((T, _EH, B // _BPW, 8, _BPW), jnp.float32),
    mesh=plsc.VectorSubcoreMesh(core_axis_name="c", subcore_axis_name="s"),
    scratch_types=[
        pltpu.VMEM((_BPW,), jnp.int32),
        pltpu.VMEM((_L, 8, 128), jnp.float32),
        pltpu.VMEM((_L, 8, 128), jnp.float32),
        pltpu.VMEM((1, _EH, 1, 8, _BPW), jnp.float32),
        pltpu.SemaphoreType.DMA,
        pltpu.SemaphoreType.DMA,
    ],
    compiler_params=pltpu.CompilerParams(
        use_tc_tiling_on_sc=True, needs_layout_passes=False
    ),
)
def _embed_repeat(x_hbm, tblt_hbm, out_hbm, idx_v, st_a, st_b, blk_v, sem, sem_b):
    wid = lax.axis_index("s") * _NC + lax.axis_index("c")
    base = wid * _BPW
    pltpu.sync_copy(x_hbm.at[pl.ds(base, _BPW)], idx_v)
    lane_iota = lax.iota(jnp.int32, _L)
    bufs = [(st_a, sem), (st_b, sem_b)]

    def group_body(g, carry):
        v = idx_v[pl.ds(g * _L, _L)]
        rt = lax.shift_right_logical(v, 7)
        lane = lax.bitwise_and(v, jnp.full((_L,), 127, jnp.int32))
        rts = [
            jnp.sum(jnp.where(lane_iota == j, rt, jnp.zeros((_L,), jnp.int32)))
            for j in range(_L)
        ]

        def fire(eh):
            st, s = bufs[eh % 2]
            return [
                pltpu.async_copy(
                    tblt_hbm.at[pl.ds(eh * 8, 8), pl.ds(rts[j] * 128, 128)],
                    st.at[j],
                    s,
                )
                for j in range(_L)
            ]

        fetches = fire(0)
        for eh in range(_EH):
            nxt = fire(eh + 1) if eh + 1 < _EH else None
            for f in fetches:
                f.wait()
            st = bufs[eh % 2][0]
            for el in range(8):
                vals = plsc.load_gather(
                    st, [lane_iota, jnp.full((_L,), el, jnp.int32), lane]
                )
                blk_v[0, eh, 0, el, pl.ds(g * _L, _L)] = vals
            fetches = nxt
        return carry

    lax.fori_loop(0, _NG, group_body, 0)
    copies = [
        pltpu.async_copy(
            blk_v,
            out_hbm.at[
                pl.ds(t, 1),
                pl.ds(0, _EH),
                pl.ds(wid, 1),
                pl.ds(0, 8),
                pl.ds(0, _BPW),
            ],
            sem,
        )
        for t in range(T)
    ]
    for c in copies:
        c.wait()


def kernel(x, ue_weight):
    out = _embed_repeat(x.astype(jnp.int32), ue_weight.T)
    return out.transpose(2, 4, 0, 1, 3).reshape(B, T, E)


# speedup vs baseline: 7.5588x; 1.0085x over previous
import functools

import jax
import jax.numpy as jnp
from jax import lax
from jax.experimental import pallas as pl
from jax.experimental.pallas import tpu as pltpu
from jax.experimental.pallas import tpu_sc as plsc

T = 100
E = 64
B = 4096

_info = plsc.get_sparse_core_info()
_NC, _NS, _L = _info.num_cores, _info.num_subcores, _info.num_lanes
_NW = _NC * _NS
_BPW = B // _NW
_EH = E // 8
_NG = _BPW // _L


@functools.partial(
    pl.kernel,
    out_type=jax.ShapeDtypeStruct((T, _EH, B // _BPW, 8, _BPW), jnp.float32),
    mesh=plsc.VectorSubcoreMesh(core_axis_name="c", subcore_axis_name="s"),
    scratch_types=[
        pltpu.VMEM((_BPW,), jnp.int32),
        pltpu.VMEM((_L, 8, 128), jnp.float32),
        pltpu.VMEM((_L, 8, 128), jnp.float32),
        pltpu.VMEM((1, _EH, 1, 8, _BPW), jnp.float32),
        pltpu.SemaphoreType.DMA,
        pltpu.SemaphoreType.DMA,
        pltpu.SemaphoreType.DMA,
    ],
    compiler_params=pltpu.CompilerParams(
        use_tc_tiling_on_sc=True, needs_layout_passes=False
    ),
)
def _embed_repeat(
    x_hbm, tblt_hbm, out_hbm, idx_v, st_a, st_b, blk_v, sem, sem_b, sem_o
):
    wid = lax.axis_index("s") * _NC + lax.axis_index("c")
    base = wid * _BPW
    pltpu.sync_copy(x_hbm.at[pl.ds(base, _BPW)], idx_v)
    lane_iota = lax.iota(jnp.int32, _L)
    bufs = [(st_a, sem), (st_b, sem_b)]

    def eh_body(eh, carry):
        def fire(g):
            v = idx_v[pl.ds(g * _L, _L)]
            rt = lax.shift_right_logical(v, 7)
            rts = [
                jnp.sum(
                    jnp.where(lane_iota == j, rt, jnp.zeros((_L,), jnp.int32))
                )
                for j in range(_L)
            ]
            st, s = bufs[g % 2]
            return [
                pltpu.async_copy(
                    tblt_hbm.at[pl.ds(eh * 8, 8), pl.ds(rts[j] * 128, 128)],
                    st.at[j],
                    s,
                )
                for j in range(_L)
            ]

        fetches = fire(0)
        for g in range(_NG):
            nxt = fire(g + 1) if g + 1 < _NG else None
            for f in fetches:
                f.wait()
            v = idx_v[pl.ds(g * _L, _L)]
            lane = lax.bitwise_and(v, jnp.full((_L,), 127, jnp.int32))
            st = bufs[g % 2][0]
            for el in range(8):
                vals = plsc.load_gather(
                    st, [lane_iota, jnp.full((_L,), el, jnp.int32), lane]
                )
                blk_v[0, eh, 0, el, pl.ds(g * _L, _L)] = vals
            fetches = nxt

        def t_body(t, c):
            pltpu.async_copy(
                blk_v.at[pl.ds(0, 1), pl.ds(eh, 1), pl.ds(0, 1), pl.ds(0, 8), pl.ds(0, _BPW)],
                out_hbm.at[
                    pl.ds(t, 1),
                    pl.ds(eh, 1),
                    pl.ds(wid, 1),
                    pl.ds(0, 8),
                    pl.ds(0, _BPW),
                ],
                sem_o,
            )
            return c

        lax.fori_loop(0, T, t_body, 0)
        return carry

    lax.fori_loop(0, _EH, eh_body, 0)
    region = out_hbm.at[
        pl.ds(0, T), pl.ds(0, _EH), pl.ds(wid, 1), pl.ds(0, 8), pl.ds(0, _BPW)
    ]
    pltpu.make_async_copy(region, region, sem_o).wait()


def kernel(x, ue_weight):
    out = _embed_repeat(x.astype(jnp.int32), ue_weight.T)
    return out.transpose(2, 4, 0, 1, 3).reshape(B, T, E)


# speedup vs baseline: 7.6226x; 1.0084x over previous
import functools

import jax
import jax.numpy as jnp
from jax import lax
from jax.experimental import pallas as pl
from jax.experimental.pallas import tpu as pltpu
from jax.experimental.pallas import tpu_sc as plsc

T = 100
E = 64
B = 4096

_info = plsc.get_sparse_core_info()
_NC, _NS, _L = _info.num_cores, _info.num_subcores, _info.num_lanes
_NW = _NC * _NS
_BPW = B // _NW
_EH = E // 8
_NG = _BPW // _L


@functools.partial(
    pl.kernel,
    out_type=jax.ShapeDtypeStruct((T, _EH, B // _BPW, 8, _BPW), jnp.float32),
    mesh=plsc.VectorSubcoreMesh(core_axis_name="c", subcore_axis_name="s"),
    scratch_types=[
        pltpu.VMEM((_BPW,), jnp.int32),
        pltpu.SMEM((_BPW,), jnp.int32),
        pltpu.VMEM((_L, 8, 128), jnp.float32),
        pltpu.VMEM((_L, 8, 128), jnp.float32),
        pltpu.VMEM((1, _EH, 1, 8, _BPW), jnp.float32),
        pltpu.SemaphoreType.DMA,
        pltpu.SemaphoreType.DMA,
        pltpu.SemaphoreType.DMA,
    ],
    compiler_params=pltpu.CompilerParams(
        use_tc_tiling_on_sc=True, needs_layout_passes=False
    ),
)
def _embed_repeat(
    x_hbm, tblt_hbm, out_hbm, idx_v, idx_s, st_a, st_b, blk_v, sem, sem_b, sem_o
):
    wid = lax.axis_index("s") * _NC + lax.axis_index("c")
    base = wid * _BPW
    pltpu.sync_copy(x_hbm.at[pl.ds(base, _BPW)], idx_v)
    lane_iota = lax.iota(jnp.int32, _L)
    bufs = [(st_a, sem), (st_b, sem_b)]

    def scal_body(g, carry):
        v = idx_v[pl.ds(g * _L, _L)]
        rt = lax.shift_right_logical(v, 7)
        for j in range(_L):
            idx_s[g * _L + j] = jnp.sum(
                jnp.where(lane_iota == j, rt, jnp.zeros((_L,), jnp.int32))
            )
        return carry

    lax.fori_loop(0, _NG, scal_body, 0)

    def eh_body(eh, carry):
        def fire(g):
            st, s = bufs[g % 2]
            return [
                pltpu.async_copy(
                    tblt_hbm.at[
                        pl.ds(eh * 8, 8),
                        pl.ds(idx_s[g * _L + j] * 128, 128),
                    ],
                    st.at[j],
                    s,
                )
                for j in range(_L)
            ]

        fetches = fire(0)
        for g in range(_NG):
            nxt = fire(g + 1) if g + 1 < _NG else None
            for f in fetches:
                f.wait()
            v = idx_v[pl.ds(g * _L, _L)]
            lane = lax.bitwise_and(v, jnp.full((_L,), 127, jnp.int32))
            st = bufs[g % 2][0]
            for el in range(8):
                vals = plsc.load_gather(
                    st, [lane_iota, jnp.full((_L,), el, jnp.int32), lane]
                )
                blk_v[0, eh, 0, el, pl.ds(g * _L, _L)] = vals
            fetches = nxt

        def t_body(t, c):
            pltpu.async_copy(
                blk_v.at[pl.ds(0, 1), pl.ds(eh, 1), pl.ds(0, 1), pl.ds(0, 8), pl.ds(0, _BPW)],
                out_hbm.at[
                    pl.ds(t, 1),
                    pl.ds(eh, 1),
                    pl.ds(wid, 1),
                    pl.ds(0, 8),
                    pl.ds(0, _BPW),
                ],
                sem_o,
            )
            return c

        lax.fori_loop(0, T, t_body, 0)
        return carry

    lax.fori_loop(0, _EH, eh_body, 0)
    region = out_hbm.at[
        pl.ds(0, T), pl.ds(0, _EH), pl.ds(wid, 1), pl.ds(0, 8), pl.ds(0, _BPW)
    ]
    pltpu.make_async_copy(region, region, sem_o).wait()


def kernel(x, ue_weight):
    out = _embed_repeat(x.astype(jnp.int32), ue_weight.T)
    return out.transpose(2, 4, 0, 1, 3).reshape(B, T, E)


# speedup vs baseline: 7.8358x; 1.0280x over previous
import functools

import jax
import jax.numpy as jnp
from jax import lax
from jax.experimental import pallas as pl
from jax.experimental.pallas import tpu as pltpu
from jax.experimental.pallas import tpu_sc as plsc

T = 100
E = 64
B = 4096

_info = plsc.get_sparse_core_info()
_NC, _NS, _L = _info.num_cores, _info.num_subcores, _info.num_lanes
_NW = _NC * _NS
_BPW = B // _NW
_EH = E // 8
_NG = _BPW // _L


@functools.partial(
    pl.kernel,
    out_type=jax.ShapeDtypeStruct((T, _EH, B // _BPW, 8, _BPW), jnp.float32),
    mesh=plsc.VectorSubcoreMesh(core_axis_name="c", subcore_axis_name="s"),
    scratch_types=[
        pltpu.VMEM((_BPW,), jnp.int32),
        pltpu.SMEM((_BPW,), jnp.int32),
        pltpu.VMEM((_L, 16, 128), jnp.float32),
        pltpu.VMEM((_L, 16, 128), jnp.float32),
        pltpu.VMEM((1, _EH, 1, 8, _BPW), jnp.float32),
        pltpu.SemaphoreType.DMA,
        pltpu.SemaphoreType.DMA,
        pltpu.SemaphoreType.DMA,
    ],
    compiler_params=pltpu.CompilerParams(
        use_tc_tiling_on_sc=True, needs_layout_passes=False
    ),
)
def _embed_repeat(
    x_hbm, tblt_hbm, out_hbm, idx_v, idx_s, st_a, st_b, blk_v, sem, sem_b, sem_o
):
    wid = lax.axis_index("s") * _NC + lax.axis_index("c")
    base = wid * _BPW
    pltpu.sync_copy(x_hbm.at[pl.ds(base, _BPW)], idx_v)
    lane_iota = lax.iota(jnp.int32, _L)
    bufs = [(st_a, sem), (st_b, sem_b)]

    def scal_body(g, carry):
        v = idx_v[pl.ds(g * _L, _L)]
        rt = lax.shift_right_logical(v, 7)
        for j in range(_L):
            idx_s[g * _L + j] = jnp.sum(
                jnp.where(lane_iota == j, rt, jnp.zeros((_L,), jnp.int32))
            )
        return carry

    lax.fori_loop(0, _NG, scal_body, 0)

    def eh_body(eh2, carry):
        def fire(g):
            st, s = bufs[g % 2]
            return [
                pltpu.async_copy(
                    tblt_hbm.at[
                        pl.ds(eh2 * 16, 16),
                        pl.ds(idx_s[g * _L + j] * 128, 128),
                    ],
                    st.at[j],
                    s,
                )
                for j in range(_L)
            ]

        fetches = fire(0)
        for g in range(_NG):
            nxt = fire(g + 1) if g + 1 < _NG else None
            for f in fetches:
                f.wait()
            v = idx_v[pl.ds(g * _L, _L)]
            lane = lax.bitwise_and(v, jnp.full((_L,), 127, jnp.int32))
            st = bufs[g % 2][0]
            for es in range(16):
                vals = plsc.load_gather(
                    st, [lane_iota, jnp.full((_L,), es, jnp.int32), lane]
                )
                blk_v[0, eh2 * 2 + es // 8, 0, es % 8, pl.ds(g * _L, _L)] = vals
            fetches = nxt

        def t_body(t, c):
            pltpu.async_copy(
                blk_v.at[
                    pl.ds(0, 1),
                    pl.ds(eh2 * 2, 2),
                    pl.ds(0, 1),
                    pl.ds(0, 8),
                    pl.ds(0, _BPW),
                ],
                out_hbm.at[
                    pl.ds(t, 1),
                    pl.ds(eh2 * 2, 2),
                    pl.ds(wid, 1),
                    pl.ds(0, 8),
                    pl.ds(0, _BPW),
                ],
                sem_o,
            )
            return c

        lax.fori_loop(0, T, t_body, 0)
        return carry

    lax.fori_loop(0, _EH // 2, eh_body, 0)
    region = out_hbm.at[
        pl.ds(0, T), pl.ds(0, _EH), pl.ds(wid, 1), pl.ds(0, 8), pl.ds(0, _BPW)
    ]
    pltpu.make_async_copy(region, region, sem_o).wait()


def kernel(x, ue_weight):
    out = _embed_repeat(x.astype(jnp.int32), ue_weight.T)
    return out.transpose(2, 4, 0, 1, 3).reshape(B, T, E)


# speedup vs baseline: 7.9514x; 1.0147x over previous
import functools

import jax
import jax.numpy as jnp
from jax import lax
from jax.experimental import pallas as pl
from jax.experimental.pallas import tpu as pltpu
from jax.experimental.pallas import tpu_sc as plsc

T = 100
E = 64
B = 4096

_info = plsc.get_sparse_core_info()
_NC, _NS, _L = _info.num_cores, _info.num_subcores, _info.num_lanes
_NW = _NC * _NS
_BPW = B // _NW
_EH = E // 8
_NG = _BPW // _L


@functools.partial(
    pl.kernel,
    out_type=jax.ShapeDtypeStruct((T, _EH, B // _BPW, 8, _BPW), jnp.float32),
    mesh=plsc.VectorSubcoreMesh(core_axis_name="c", subcore_axis_name="s"),
    scratch_types=[
        pltpu.VMEM((_BPW,), jnp.int32),
        pltpu.SMEM((_BPW,), jnp.int32),
        pltpu.VMEM((_L, 16, 128), jnp.float32),
        pltpu.VMEM((_L, 16, 128), jnp.float32),
        pltpu.VMEM((_L, 16, 128), jnp.float32),
        pltpu.VMEM((1, _EH, 1, 8, _BPW), jnp.float32),
        pltpu.SemaphoreType.DMA,
        pltpu.SemaphoreType.DMA,
        pltpu.SemaphoreType.DMA,
        pltpu.SemaphoreType.DMA,
    ],
    compiler_params=pltpu.CompilerParams(
        use_tc_tiling_on_sc=True, needs_layout_passes=False
    ),
)
def _embed_repeat(
    x_hbm,
    tblt_hbm,
    out_hbm,
    idx_v,
    idx_s,
    st_a,
    st_b,
    st_c,
    blk_v,
    sem,
    sem_b,
    sem_c,
    sem_o,
):
    wid = lax.axis_index("s") * _NC + lax.axis_index("c")
    base = wid * _BPW
    pltpu.sync_copy(x_hbm.at[pl.ds(base, _BPW)], idx_v)
    lane_iota = lax.iota(jnp.int32, _L)
    bufs = [(st_a, sem), (st_b, sem_b), (st_c, sem_c)]

    def scal_body(g, carry):
        v = idx_v[pl.ds(g * _L, _L)]
        rt = lax.shift_right_logical(v, 7)
        for j in range(_L):
            idx_s[g * _L + j] = jnp.sum(
                jnp.where(lane_iota == j, rt, jnp.zeros((_L,), jnp.int32))
            )
        return carry

    lax.fori_loop(0, _NG, scal_body, 0)

    def eh_body(eh2, carry):
        def fire(g):
            st, s = bufs[g % 3]
            return [
                pltpu.async_copy(
                    tblt_hbm.at[
                        pl.ds(eh2 * 16, 16),
                        pl.ds(idx_s[g * _L + j] * 128, 128),
                    ],
                    st.at[j],
                    s,
                )
                for j in range(_L)
            ]

        inflight = [fire(0), fire(1)]
        for g in range(_NG):
            if g + 2 < _NG:
                inflight.append(fire(g + 2))
            for f in inflight.pop(0):
                f.wait()
            v = idx_v[pl.ds(g * _L, _L)]
            lane = lax.bitwise_and(v, jnp.full((_L,), 127, jnp.int32))
            st = bufs[g % 3][0]
            for es in range(16):
                vals = plsc.load_gather(
                    st, [lane_iota, jnp.full((_L,), es, jnp.int32), lane]
                )
                blk_v[0, eh2 * 2 + es // 8, 0, es % 8, pl.ds(g * _L, _L)] = vals

        def t_body(t, c):
            pltpu.async_copy(
                blk_v.at[
                    pl.ds(0, 1),
                    pl.ds(eh2 * 2, 2),
                    pl.ds(0, 1),
                    pl.ds(0, 8),
                    pl.ds(0, _BPW),
                ],
                out_hbm.at[
                    pl.ds(t, 1),
                    pl.ds(eh2 * 2, 2),
                    pl.ds(wid, 1),
                    pl.ds(0, 8),
                    pl.ds(0, _BPW),
                ],
                sem_o,
            )
            return c

        lax.fori_loop(0, T, t_body, 0)
        return carry

    lax.fori_loop(0, _EH // 2, eh_body, 0)
    region = out_hbm.at[
        pl.ds(0, T), pl.ds(0, _EH), pl.ds(wid, 1), pl.ds(0, 8), pl.ds(0, _BPW)
    ]
    pltpu.make_async_copy(region, region, sem_o).wait()


def kernel(x, ue_weight):
    out = _embed_repeat(x.astype(jnp.int32), ue_weight.T)
    return out.transpose(2, 4, 0, 1, 3).reshape(B, T, E)
